# baseline (speedup 1.0000x reference)
import jax
import jax.numpy as jnp
from jax.experimental import pallas as pl
from jax.experimental.pallas import tpu as pltpu

_ALPHA = 16.0
_ER = 32
_E = 8
_CT = 512
_IDEPTH = 4
_ODEPTH = 3


def _molora_manual(x_hbm, w_ref, b_ref, exp_ref, bcat_ref, out_hbm,
                   in_bufs, in_sems, out_bufs, out_sems):
    n_chunks = x_hbm.shape[0] // (2 * _CT)
    core = pl.program_id(0)
    base = core * (n_chunks * _CT)

    def in_cp(k):
        rows = base + k * _CT
        return pltpu.make_async_copy(
            x_hbm.at[pl.ds(rows, _CT), :],
            in_bufs.at[k % _IDEPTH],
            in_sems.at[k % _IDEPTH])

    def out_cp(k):
        rows = base + k * _CT
        return pltpu.make_async_copy(
            out_bufs.at[k % _ODEPTH],
            out_hbm.at[pl.ds(rows, _CT), :],
            out_sems.at[k % _ODEPTH])

    for k in range(_IDEPTH):
        if k < n_chunks:
            in_cp(k).start()

    for k in range(n_chunks):
        in_cp(k).wait()
        if k >= _ODEPTH:
            out_cp(k - _ODEPTH).wait()
        xb = in_bufs[k % _IDEPTH]
        y = jnp.dot(xb, w_ref[...], preferred_element_type=jnp.float32)
        ax = y[:, :_ER]
        logits = y[:, _ER:_ER + _E] + b_ref[...]
        m = jnp.max(logits, axis=-1, keepdims=True)
        ex = jnp.exp(logits - m)
        probs = ex / jnp.sum(ex, axis=-1, keepdims=True)
        probs_er = jnp.dot(probs, exp_ref[...],
                           preferred_element_type=jnp.float32)
        out_bufs[k % _ODEPTH] = jnp.dot(ax * probs_er, bcat_ref[...],
                                        preferred_element_type=jnp.float32)
        out_cp(k).start(priority=1)
        if k + _IDEPTH < n_chunks:
            in_cp(k + _IDEPTH).start()

    for k in range(max(n_chunks - _ODEPTH, 0), n_chunks):
        out_cp(k).wait()


def kernel(x, lora_A, lora_B, router_w, router_b):
    b, s, d = x.shape
    e, _, r = lora_A.shape
    tokens = b * s

    x2 = x.reshape(tokens, d)
    a_cat = lora_A.transpose(1, 0, 2).reshape(d, e * r)
    w_fused = jnp.concatenate([a_cat, router_w], axis=1)
    b_cat = lora_B.reshape(e * r, d) * (_ALPHA / r)
    expand = jnp.repeat(jnp.eye(e, dtype=jnp.float32), r, axis=1)
    bias = router_b.reshape(1, e)

    out = pl.pallas_call(
        _molora_manual,
        grid=(2,),
        in_specs=[
            pl.BlockSpec(memory_space=pl.ANY),
            pl.BlockSpec((d, e * r + e), lambda i: (0, 0)),
            pl.BlockSpec((1, e), lambda i: (0, 0)),
            pl.BlockSpec((e, e * r), lambda i: (0, 0)),
            pl.BlockSpec((e * r, d), lambda i: (0, 0)),
        ],
        out_specs=pl.BlockSpec(memory_space=pl.ANY),
        out_shape=jax.ShapeDtypeStruct((tokens, d), jnp.float32),
        scratch_shapes=[
            pltpu.VMEM((_IDEPTH, _CT, d), jnp.float32),
            pltpu.SemaphoreType.DMA((_IDEPTH,)),
            pltpu.VMEM((_ODEPTH, _CT, d), jnp.float32),
            pltpu.SemaphoreType.DMA((_ODEPTH,)),
        ],
        compiler_params=pltpu.CompilerParams(
            dimension_semantics=("parallel",),
            vmem_limit_bytes=60 * 1024 * 1024,
        ),
    )(x2, w_fused, bias, expand, b_cat)
    return out.reshape(b, s, d)


# speedup vs baseline: 1.0877x; 1.0877x over previous
import jax
import jax.numpy as jnp
from jax.experimental import pallas as pl
from jax.experimental.pallas import tpu as pltpu

_ALPHA = 16.0
_E = 8
_ER = 32
_T1 = 1024
_T2 = 1024


def _pass1(x_ref, w_ref, b_ref, exp_ref, scaled_ref):
    y = jnp.dot(x_ref[...], w_ref[...], preferred_element_type=jnp.float32)
    ax = y[:, :_ER]
    logits = y[:, _ER:_ER + _E] + b_ref[...]
    m = jnp.max(logits, axis=-1, keepdims=True)
    ex = jnp.exp(logits - m)
    probs = ex / jnp.sum(ex, axis=-1, keepdims=True)
    probs_er = jnp.dot(probs, exp_ref[...], preferred_element_type=jnp.float32)
    scaled_ref[...] = (ax * probs_er).astype(jnp.bfloat16)


def _pass2(scaled_ref, bcat_ref, out_ref):
    out_ref[...] = jnp.dot(scaled_ref[...], bcat_ref[...],
                           preferred_element_type=jnp.float32)


def kernel(x, lora_A, lora_B, router_w, router_b):
    b, s, d = x.shape
    e, _, r = lora_A.shape
    tokens = b * s

    x2 = x.reshape(tokens, d)
    a_cat = lora_A.transpose(1, 0, 2).reshape(d, e * r)
    w_fused = jnp.concatenate([a_cat, router_w], axis=1)
    b_cat = lora_B.reshape(e * r, d) * (_ALPHA / r)
    expand = jnp.repeat(jnp.eye(e, dtype=jnp.float32), r, axis=1)
    bias = router_b.reshape(1, e)

    scaled = pl.pallas_call(
        _pass1,
        grid=(tokens // _T1,),
        in_specs=[
            pl.BlockSpec((_T1, d), lambda i: (i, 0)),
            pl.BlockSpec((d, e * r + e), lambda i: (0, 0)),
            pl.BlockSpec((1, e), lambda i: (0, 0)),
            pl.BlockSpec((e, e * r), lambda i: (0, 0)),
        ],
        out_specs=pl.BlockSpec((_T1, e * r), lambda i: (i, 0)),
        out_shape=jax.ShapeDtypeStruct((tokens, e * r), jnp.bfloat16),
        compiler_params=pltpu.CompilerParams(
            dimension_semantics=("parallel",),
            vmem_limit_bytes=60 * 1024 * 1024,
        ),
    )(x2, w_fused, bias, expand)

    out = pl.pallas_call(
        _pass2,
        grid=(tokens // _T2,),
        in_specs=[
            pl.BlockSpec((_T2, e * r), lambda i: (i, 0)),
            pl.BlockSpec((e * r, d), lambda i: (0, 0)),
        ],
        out_specs=pl.BlockSpec((_T2, d), lambda i: (i, 0)),
        out_shape=jax.ShapeDtypeStruct((tokens, d), jnp.float32),
        compiler_params=pltpu.CompilerParams(
            dimension_semantics=("parallel",),
            vmem_limit_bytes=60 * 1024 * 1024,
        ),
    )(scaled, b_cat)
    return out.reshape(b, s, d)


# speedup vs baseline: 1.1077x; 1.0184x over previous
import jax
import jax.numpy as jnp
from jax.experimental import pallas as pl
from jax.experimental.pallas import tpu as pltpu

_ALPHA = 16.0
_E = 8
_ER = 32
_T1 = 2048
_T2 = 1024


def _pass1(x_ref, w_ref, b_ref, exp_ref, scaled_ref):
    y = jnp.dot(x_ref[...], w_ref[...], preferred_element_type=jnp.float32)
    ax = y[:, :_ER]
    logits = y[:, _ER:_ER + _E] + b_ref[...]
    m = jnp.max(logits, axis=-1, keepdims=True)
    ex = jnp.exp(logits - m)
    probs = ex / jnp.sum(ex, axis=-1, keepdims=True)
    probs_er = jnp.dot(probs, exp_ref[...], preferred_element_type=jnp.float32)
    scaled_ref[...] = ax * probs_er


def _pass2(scaled_ref, bcat_ref, out_ref):
    out_ref[...] = jnp.dot(scaled_ref[...], bcat_ref[...],
                           preferred_element_type=jnp.float32)


def kernel(x, lora_A, lora_B, router_w, router_b):
    b, s, d = x.shape
    e, _, r = lora_A.shape
    tokens = b * s

    x2 = x.reshape(tokens, d)
    a_cat = lora_A.transpose(1, 0, 2).reshape(d, e * r)
    w_fused = jnp.concatenate([a_cat, router_w], axis=1)
    b_cat = lora_B.reshape(e * r, d) * (_ALPHA / r)
    expand = jnp.repeat(jnp.eye(e, dtype=jnp.float32), r, axis=1)
    bias = router_b.reshape(1, e)

    scaled = pl.pallas_call(
        _pass1,
        grid=(tokens // _T1,),
        in_specs=[
            pl.BlockSpec((_T1, d), lambda i: (i, 0)),
            pl.BlockSpec((d, e * r + e), lambda i: (0, 0)),
            pl.BlockSpec((1, e), lambda i: (0, 0)),
            pl.BlockSpec((e, e * r), lambda i: (0, 0)),
        ],
        out_specs=pl.BlockSpec((_T1, e * r), lambda i: (i, 0)),
        out_shape=jax.ShapeDtypeStruct((tokens, e * r), jnp.float32),
        compiler_params=pltpu.CompilerParams(
            dimension_semantics=("parallel",),
            vmem_limit_bytes=60 * 1024 * 1024,
        ),
    )(x2, w_fused, bias, expand)

    out = pl.pallas_call(
        _pass2,
        grid=(tokens // _T2,),
        in_specs=[
            pl.BlockSpec((_T2, e * r), lambda i: (i, 0)),
            pl.BlockSpec((e * r, d), lambda i: (0, 0)),
        ],
        out_specs=pl.BlockSpec((_T2, d), lambda i: (i, 0)),
        out_shape=jax.ShapeDtypeStruct((tokens, d), jnp.float32),
        compiler_params=pltpu.CompilerParams(
            dimension_semantics=("parallel",),
            vmem_limit_bytes=60 * 1024 * 1024,
        ),
    )(scaled, b_cat)
    return out.reshape(b, s, d)
